# baseline (speedup 1.0000x reference)
import functools

import jax
import jax.numpy as jnp
from jax import lax
from jax.experimental import pallas as pl
from jax.experimental.pallas import tpu as pltpu
from jax.experimental.pallas import tpu_sc as plsc

B, S, D, DFF = 4, 4096, 2048, 8192
K = S // 2
N = B * S
NSEL = B * K
NC, NS = 2, 16
NW = NC * NS
L = 16


_ROUT_BLK = 1024


def _router_body(x_ref, w_ref, b_ref, out_ref):
    x = x_ref[...].astype(jnp.bfloat16).astype(jnp.float32)
    w = w_ref[...].astype(jnp.bfloat16).astype(jnp.float32)
    out_ref[...] = jnp.sum(x * w, axis=1, keepdims=True) + b_ref[0]


def _router_logits(h_flat, w, rb):
    return pl.pallas_call(
        _router_body,
        grid=(N // _ROUT_BLK,),
        in_specs=[
            pl.BlockSpec((_ROUT_BLK, D), lambda i: (i, 0)),
            pl.BlockSpec((1, D), lambda i: (0, 0)),
            pl.BlockSpec(memory_space=pltpu.SMEM),
        ],
        out_specs=pl.BlockSpec((_ROUT_BLK, 1), lambda i: (i, 0)),
        out_shape=jax.ShapeDtypeStruct((N, 1), jnp.float32),
        compiler_params=pltpu.CompilerParams(
            dimension_semantics=("arbitrary",),
        ),
    )(h_flat, w.reshape(1, D), rb)


_NV = S // L


def _select_body(lg_hbm, sel_i_hbm, sel_w_hbm,
                 lg_v, key_v, sel_i_v, sel_w_v):
    wid = lax.axis_index("s") * NC + lax.axis_index("c")

    @pl.when(wid < B)
    def _():
        row = wid
        pltpu.sync_copy(lg_hbm.at[pl.ds(row * S, S)], lg_v)

        shift31 = jnp.full((L,), 31, jnp.uint32)
        signbit = jnp.full((L,), 0x80000000, jnp.uint32)
        zero_u = jnp.zeros((L,), jnp.uint32)
        one_f = jnp.ones((L,), jnp.float32)

        def keys_loop(j, _):
            v = lg_v[pl.ds(j * L, L)]
            u = lax.bitcast_convert_type(v, jnp.uint32)
            neg = (u >> shift31) != zero_u
            key = jnp.where(neg, ~u, u | signbit)
            key_v[pl.ds(j * L, L)] = key
            return 0

        lax.fori_loop(0, _NV, keys_loop, 0)

        def count_ge(t):
            tv = lax.broadcast(t, (L,))

            def body(j, acc):
                kv = key_v[pl.ds(j * L, L)]
                return acc + (kv >= tv).astype(jnp.int32)

            acc = lax.fori_loop(0, _NV, body, jnp.zeros((L,), jnp.int32))
            return jnp.sum(acc)

        def bit_step(t, prefix):
            bit = jnp.uint32(31) - t.astype(jnp.uint32)
            cand = prefix | (jnp.uint32(1) << bit)
            cnt = count_ge(cand)
            return jnp.where(cnt >= K, cand, prefix)

        thresh = lax.fori_loop(0, 32, bit_step, jnp.uint32(0))
        thresh_v = lax.broadcast(thresh, (L,))

        def count_gt_body(j, acc):
            kv = key_v[pl.ds(j * L, L)]
            return acc + (kv > thresh_v).astype(jnp.int32)

        n_gt = jnp.sum(lax.fori_loop(0, _NV, count_gt_body,
                                     jnp.zeros((L,), jnp.int32)))
        quota = K - n_gt
        quota_v = lax.broadcast(quota, (L,))

        lane = lax.iota(jnp.int32, L)

        def compact(j, carry):
            sel_pos, eq_taken = carry
            kv = key_v[pl.ds(j * L, L)]
            v = lg_v[pl.ds(j * L, L)]
            m_gt = kv > thresh_v
            m_eq = kv == thresh_v
            eqc = plsc.cumsum(m_eq.astype(jnp.int32))
            take_eq = m_eq & ((lax.broadcast(eq_taken, (L,)) + eqc) <= quota_v)
            m_sel = m_gt | take_eq
            ids = lane + lax.broadcast(row * S + j * L, (L,))
            sig = one_f / (one_f + jnp.exp(-v))
            plsc.store_compressed(sel_i_v.at[pl.ds(sel_pos, L)], ids, mask=m_sel)
            plsc.store_compressed(sel_w_v.at[pl.ds(sel_pos, L)], sig, mask=m_sel)
            n_sel = jnp.sum(m_sel.astype(jnp.int32))
            n_eq = jnp.sum(take_eq.astype(jnp.int32))
            return sel_pos + n_sel, eq_taken + n_eq

        lax.fori_loop(0, _NV, compact, (jnp.int32(0), jnp.int32(0)))

        pltpu.sync_copy(sel_i_v.at[pl.ds(0, K)], sel_i_hbm.at[pl.ds(row * K, K)])
        pltpu.sync_copy(sel_w_v.at[pl.ds(0, K)], sel_w_hbm.at[pl.ds(row * K, K)])


def _select(logits_flat):
    f = pl.kernel(
        _select_body,
        out_type=(
            jax.ShapeDtypeStruct((NSEL,), jnp.int32),
            jax.ShapeDtypeStruct((NSEL,), jnp.float32),
        ),
        mesh=plsc.VectorSubcoreMesh(core_axis_name="c", subcore_axis_name="s"),
        scratch_types=[
            pltpu.VMEM((S,), jnp.float32),
            pltpu.VMEM((S,), jnp.uint32),
            pltpu.VMEM((K + L,), jnp.int32),
            pltpu.VMEM((K + L,), jnp.float32),
        ],
        compiler_params=pltpu.CompilerParams(needs_layout_passes=False),
    )
    return f(logits_flat)


_GCH = 16
_RPW = NSEL // NW
_NCH = _RPW // _GCH


def _gather_body(h_hbm, idx_hbm, out_hbm, *rest):
    idx = rest[0:3]
    buf = rest[3:6]
    sg = rest[6:9]
    sw = rest[9:12]
    wid = lax.axis_index("c") * NS + lax.axis_index("s")
    base = wid * _RPW

    def start_gather(c, b):
        pltpu.sync_copy(idx_hbm.at[pl.ds(base + c * _GCH, _GCH)], idx[b])
        pltpu.async_copy(h_hbm.at[idx[b]], buf[b], sg[b])

    start_gather(0, 0)
    start_gather(1, 1)
    for c in range(_NCH):
        b = c % 3
        pltpu.make_async_copy(h_hbm.at[idx[b]], buf[b], sg[b]).wait()
        pltpu.async_copy(buf[b], out_hbm.at[pl.ds(base + c * _GCH, _GCH)],
                         sw[b])
        n = c + 2
        if n < _NCH:
            bn = n % 3
            if n >= 3:
                pltpu.make_async_copy(
                    buf[bn], out_hbm.at[pl.ds(base + (n - 3) * _GCH, _GCH)],
                    sw[bn]).wait()
            start_gather(n, bn)
    for t in range(_NCH - 3, _NCH):
        pltpu.make_async_copy(
            buf[t % 3], out_hbm.at[pl.ds(base + t * _GCH, _GCH)],
            sw[t % 3]).wait()


def _gather(h_flat, sel_idx):
    f = pl.kernel(
        _gather_body,
        out_type=jax.ShapeDtypeStruct((NSEL, D), jnp.float32),
        mesh=plsc.VectorSubcoreMesh(core_axis_name="c", subcore_axis_name="s"),
        scratch_types=(
            [pltpu.VMEM((_GCH,), jnp.int32)] * 3
            + [pltpu.VMEM((_GCH, D), jnp.float32)] * 3
            + [pltpu.SemaphoreType.DMA] * 6
        ),
        compiler_params=pltpu.CompilerParams(needs_layout_passes=False),
    )
    return f(h_flat, sel_idx)


_M = 1024
_FC = 1024
_NJ = DFF // _FC


def _ffn_body(x_ref, w1_ref, b1_ref, w2_ref, b2_ref, sw_ref, out_ref, xb_ref):
    j = pl.program_id(1)

    @pl.when(j == 0)
    def _():
        xb_ref[...] = x_ref[...].astype(jnp.bfloat16)

    h = jnp.dot(xb_ref[...], w1_ref[...], preferred_element_type=jnp.float32)
    h = h + b1_ref[...]
    h = jax.nn.gelu(h)
    c = jnp.dot(h.astype(jnp.bfloat16), w2_ref[...],
                preferred_element_type=jnp.float32)

    @pl.when(j == 0)
    def _():
        out_ref[...] = c

    @pl.when(j != 0)
    def _():
        out_ref[...] += c

    @pl.when(j == _NJ - 1)
    def _():
        out_ref[...] = x_ref[...] + sw_ref[...] * (out_ref[...] + b2_ref[...])


def _ffn(xg, w1b, b1, w2b, b2, sel_w):
    return pl.pallas_call(
        _ffn_body,
        grid=(NSEL // _M, _NJ),
        in_specs=[
            pl.BlockSpec((_M, D), lambda i, j: (i, 0)),
            pl.BlockSpec((D, _FC), lambda i, j: (0, j)),
            pl.BlockSpec((1, _FC), lambda i, j: (0, j)),
            pl.BlockSpec((_FC, D), lambda i, j: (j, 0)),
            pl.BlockSpec((1, D), lambda i, j: (0, 0)),
            pl.BlockSpec((_M, 1), lambda i, j: (i, 0)),
        ],
        out_specs=pl.BlockSpec((_M, D), lambda i, j: (i, 0)),
        out_shape=jax.ShapeDtypeStruct((NSEL, D), jnp.float32),
        scratch_shapes=[pltpu.VMEM((_M, D), jnp.bfloat16)],
        compiler_params=pltpu.CompilerParams(
            dimension_semantics=("parallel", "arbitrary"),
            vmem_limit_bytes=100 * 1024 * 1024,
        ),
    )(xg, w1b, b1.reshape(1, DFF), w2b, b2.reshape(1, D),
      sel_w.reshape(NSEL, 1))


def _scatter_body(yg_hbm, sel_hbm, out_hbm, *rest):
    idx = rest[0:3]
    buf = rest[3:6]
    sr = rest[6:9]
    sw = rest[9:12]
    wid = lax.axis_index("c") * NS + lax.axis_index("s")
    base = wid * _RPW

    def start_read(c, b):
        pltpu.async_copy(yg_hbm.at[pl.ds(base + c * _GCH, _GCH)], buf[b],
                         sr[b])

    start_read(0, 0)
    start_read(1, 1)
    for c in range(_NCH):
        b = c % 3
        pltpu.make_async_copy(yg_hbm.at[pl.ds(base + c * _GCH, _GCH)], buf[b],
                              sr[b]).wait()
        pltpu.sync_copy(sel_hbm.at[pl.ds(base + c * _GCH, _GCH)], idx[b])
        pltpu.async_copy(buf[b], out_hbm.at[idx[b]], sw[b])
        n = c + 2
        if n < _NCH:
            bn = n % 3
            if n >= 3:
                pltpu.make_async_copy(buf[bn], out_hbm.at[idx[bn]],
                                      sw[bn]).wait()
            start_read(n, bn)
    for t in range(_NCH - 3, _NCH):
        pltpu.make_async_copy(buf[t % 3], out_hbm.at[idx[t % 3]],
                              sw[t % 3]).wait()


def _scatter_into(out_ref, yg, sel_idx):
    f = pl.kernel(
        _scatter_body,
        out_type=(),
        mesh=plsc.VectorSubcoreMesh(core_axis_name="c", subcore_axis_name="s"),
        scratch_types=(
            [pltpu.VMEM((_GCH,), jnp.int32)] * 3
            + [pltpu.VMEM((_GCH, D), jnp.float32)] * 3
            + [pltpu.SemaphoreType.DMA] * 6
        ),
        compiler_params=pltpu.CompilerParams(needs_layout_passes=False),
    )
    f(yg, sel_idx, out_ref)


def kernel(hidden_states, router_weight, router_bias, W1, b1, W2, b2):
    h_flat = hidden_states.reshape(N, D)
    logits = _router_logits(h_flat, router_weight, router_bias).reshape(N)
    sel_idx, sel_w = _select(logits)
    xg = _gather(h_flat, sel_idx)
    yg = _ffn(xg, W1.astype(jnp.bfloat16), b1, W2.astype(jnp.bfloat16), b2,
              sel_w)
    out_ref = jax.new_ref(h_flat)
    _scatter_into(out_ref, yg, sel_idx)
    return out_ref[...].reshape(B, S, D)


# speedup vs baseline: 1.1306x; 1.1306x over previous
import functools

import jax
import jax.numpy as jnp
from jax import lax
from jax.experimental import pallas as pl
from jax.experimental.pallas import tpu as pltpu
from jax.experimental.pallas import tpu_sc as plsc

B, S, D, DFF = 4, 4096, 2048, 8192
K = S // 2
N = B * S
NSEL = B * K
NC, NS = 2, 16
NW = NC * NS
L = 16


_ROUT_BLK = 1024


def _router_body(x_ref, w_ref, b_ref, out_ref, base_ref):
    xv = x_ref[...]
    base_ref[...] = xv
    x = xv.astype(jnp.bfloat16).astype(jnp.float32)
    w = w_ref[...].astype(jnp.bfloat16).astype(jnp.float32)
    out_ref[...] = jnp.sum(x * w, axis=1, keepdims=True) + b_ref[0]


def _router_logits(h_flat, w, rb):
    return pl.pallas_call(
        _router_body,
        grid=(N // _ROUT_BLK,),
        in_specs=[
            pl.BlockSpec((_ROUT_BLK, D), lambda i: (i, 0)),
            pl.BlockSpec((1, D), lambda i: (0, 0)),
            pl.BlockSpec(memory_space=pltpu.SMEM),
        ],
        out_specs=[
            pl.BlockSpec((_ROUT_BLK, 1), lambda i: (i, 0)),
            pl.BlockSpec((_ROUT_BLK, D), lambda i: (i, 0)),
        ],
        out_shape=[
            jax.ShapeDtypeStruct((N, 1), jnp.float32),
            jax.ShapeDtypeStruct((N, D), jnp.float32),
        ],
        compiler_params=pltpu.CompilerParams(
            dimension_semantics=("arbitrary",),
        ),
    )(h_flat, w.reshape(1, D), rb)


_NV = S // L


def _select_body(lg_hbm, sel_i_hbm, sel_w_hbm,
                 lg_v, key_v, sel_i_v, sel_w_v):
    wid = lax.axis_index("s") * NC + lax.axis_index("c")

    @pl.when(wid < B)
    def _():
        row = wid
        pltpu.sync_copy(lg_hbm.at[pl.ds(row * S, S)], lg_v)

        shift31 = jnp.full((L,), 31, jnp.uint32)
        signbit = jnp.full((L,), 0x80000000, jnp.uint32)
        zero_u = jnp.zeros((L,), jnp.uint32)
        one_f = jnp.ones((L,), jnp.float32)

        def keys_loop(j, _):
            v = lg_v[pl.ds(j * L, L)]
            u = lax.bitcast_convert_type(v, jnp.uint32)
            neg = (u >> shift31) != zero_u
            key = jnp.where(neg, ~u, u | signbit)
            key_v[pl.ds(j * L, L)] = key
            return 0

        lax.fori_loop(0, _NV, keys_loop, 0)

        def count_ge(t):
            tv = lax.broadcast(t, (L,))

            def body(j, acc):
                kv = key_v[pl.ds(j * L, L)]
                return acc + (kv >= tv).astype(jnp.int32)

            acc = lax.fori_loop(0, _NV, body, jnp.zeros((L,), jnp.int32))
            return jnp.sum(acc)

        def bit_step(t, prefix):
            bit = jnp.uint32(31) - t.astype(jnp.uint32)
            cand = prefix | (jnp.uint32(1) << bit)
            cnt = count_ge(cand)
            return jnp.where(cnt >= K, cand, prefix)

        thresh = lax.fori_loop(0, 32, bit_step, jnp.uint32(0))
        thresh_v = lax.broadcast(thresh, (L,))

        def count_gt_body(j, acc):
            kv = key_v[pl.ds(j * L, L)]
            return acc + (kv > thresh_v).astype(jnp.int32)

        n_gt = jnp.sum(lax.fori_loop(0, _NV, count_gt_body,
                                     jnp.zeros((L,), jnp.int32)))
        quota = K - n_gt
        quota_v = lax.broadcast(quota, (L,))

        lane = lax.iota(jnp.int32, L)

        def compact(j, carry):
            sel_pos, eq_taken = carry
            kv = key_v[pl.ds(j * L, L)]
            v = lg_v[pl.ds(j * L, L)]
            m_gt = kv > thresh_v
            m_eq = kv == thresh_v
            eqc = plsc.cumsum(m_eq.astype(jnp.int32))
            take_eq = m_eq & ((lax.broadcast(eq_taken, (L,)) + eqc) <= quota_v)
            m_sel = m_gt | take_eq
            ids = lane + lax.broadcast(row * S + j * L, (L,))
            sig = one_f / (one_f + jnp.exp(-v))
            plsc.store_compressed(sel_i_v.at[pl.ds(sel_pos, L)], ids, mask=m_sel)
            plsc.store_compressed(sel_w_v.at[pl.ds(sel_pos, L)], sig, mask=m_sel)
            n_sel = jnp.sum(m_sel.astype(jnp.int32))
            n_eq = jnp.sum(take_eq.astype(jnp.int32))
            return sel_pos + n_sel, eq_taken + n_eq

        lax.fori_loop(0, _NV, compact, (jnp.int32(0), jnp.int32(0)))

        pltpu.sync_copy(sel_i_v.at[pl.ds(0, K)], sel_i_hbm.at[pl.ds(row * K, K)])
        pltpu.sync_copy(sel_w_v.at[pl.ds(0, K)], sel_w_hbm.at[pl.ds(row * K, K)])


def _select(logits_flat):
    f = pl.kernel(
        _select_body,
        out_type=(
            jax.ShapeDtypeStruct((NSEL,), jnp.int32),
            jax.ShapeDtypeStruct((NSEL,), jnp.float32),
        ),
        mesh=plsc.VectorSubcoreMesh(core_axis_name="c", subcore_axis_name="s"),
        scratch_types=[
            pltpu.VMEM((S,), jnp.float32),
            pltpu.VMEM((S,), jnp.uint32),
            pltpu.VMEM((K + L,), jnp.int32),
            pltpu.VMEM((K + L,), jnp.float32),
        ],
        compiler_params=pltpu.CompilerParams(needs_layout_passes=False),
    )
    return f(logits_flat)


_GCH = 16
_RPW = NSEL // NW
_NCH = _RPW // _GCH


def _gather_body(h_hbm, idx_hbm, out_hbm, *rest):
    idx = rest[0:3]
    buf = rest[3:6]
    sg = rest[6:9]
    sw = rest[9:12]
    wid = lax.axis_index("c") * NS + lax.axis_index("s")
    base = wid * _RPW

    def start_gather(c, b):
        pltpu.sync_copy(idx_hbm.at[pl.ds(base + c * _GCH, _GCH)], idx[b])
        pltpu.async_copy(h_hbm.at[idx[b]], buf[b], sg[b])

    start_gather(0, 0)
    start_gather(1, 1)
    for c in range(_NCH):
        b = c % 3
        pltpu.make_async_copy(h_hbm.at[idx[b]], buf[b], sg[b]).wait()
        pltpu.async_copy(buf[b], out_hbm.at[pl.ds(base + c * _GCH, _GCH)],
                         sw[b])
        n = c + 2
        if n < _NCH:
            bn = n % 3
            if n >= 3:
                pltpu.make_async_copy(
                    buf[bn], out_hbm.at[pl.ds(base + (n - 3) * _GCH, _GCH)],
                    sw[bn]).wait()
            start_gather(n, bn)
    for t in range(_NCH - 3, _NCH):
        pltpu.make_async_copy(
            buf[t % 3], out_hbm.at[pl.ds(base + t * _GCH, _GCH)],
            sw[t % 3]).wait()


def _gather(h_flat, sel_idx):
    f = pl.kernel(
        _gather_body,
        out_type=jax.ShapeDtypeStruct((NSEL, D), jnp.float32),
        mesh=plsc.VectorSubcoreMesh(core_axis_name="c", subcore_axis_name="s"),
        scratch_types=(
            [pltpu.VMEM((_GCH,), jnp.int32)] * 3
            + [pltpu.VMEM((_GCH, D), jnp.float32)] * 3
            + [pltpu.SemaphoreType.DMA] * 6
        ),
        compiler_params=pltpu.CompilerParams(needs_layout_passes=False),
    )
    return f(h_flat, sel_idx)


_M = 512
_FC = 2048
_FT = 512
_NJ = DFF // _FC


def _ffn_body(xb_ref, w1_ref, b1_ref, w2_ref, b2_ref, sw_ref, out_ref):
    jj = pl.program_id(1)
    h = jnp.dot(xb_ref[...].astype(jnp.bfloat16), w1_ref[...],
                preferred_element_type=jnp.float32)
    g = jax.nn.gelu((h + b1_ref[...]).astype(jnp.bfloat16))
    c = jnp.dot(g, w2_ref[...], preferred_element_type=jnp.float32)

    @pl.when(jj == 0)
    def _():
        out_ref[...] = c

    @pl.when(jj != 0)
    def _():
        out_ref[...] += c

    @pl.when(jj == _NJ - 1)
    def _():
        out_ref[...] = (xb_ref[...]
                        + sw_ref[...] * (out_ref[...] + b2_ref[...]))


def _clamp(v, lo, hi):
    return jnp.maximum(lo, jnp.minimum(v, hi))


def _ffn(xg, w1b, b1, w2b, b2, sel_w):
    return pl.pallas_call(
        _ffn_body,
        grid=(NSEL // _M, _NJ),
        in_specs=[
            pl.BlockSpec((_M, D), lambda i, jj: (i, 0)),
            pl.BlockSpec((D, _FC), lambda i, jj: (0, jj)),
            pl.BlockSpec((1, _FC), lambda i, jj: (0, jj)),
            pl.BlockSpec((_FC, D), lambda i, jj: (jj, 0)),
            pl.BlockSpec((1, D), lambda i, jj: (0, 0)),
            pl.BlockSpec((_M, 1), lambda i, jj: (i, 0)),
        ],
        out_specs=pl.BlockSpec((_M, D), lambda i, jj: (i, 0)),
        out_shape=jax.ShapeDtypeStruct((NSEL, D), jnp.float32),
        compiler_params=pltpu.CompilerParams(
            dimension_semantics=("parallel", "arbitrary"),
            vmem_limit_bytes=100 * 1024 * 1024,
        ),
    )(xg, w1b, b1.reshape(1, DFF), w2b, b2.reshape(1, D),
      sel_w.reshape(NSEL, 1))


def _scatter_body(yg_hbm, sel_hbm, out_hbm, *rest):
    idx = rest[0:3]
    buf = rest[3:6]
    sr = rest[6:9]
    sw = rest[9:12]
    wid = lax.axis_index("c") * NS + lax.axis_index("s")
    base = wid * _RPW

    def start_read(c, b):
        pltpu.async_copy(yg_hbm.at[pl.ds(base + c * _GCH, _GCH)], buf[b],
                         sr[b])

    start_read(0, 0)
    start_read(1, 1)
    for c in range(_NCH):
        b = c % 3
        pltpu.make_async_copy(yg_hbm.at[pl.ds(base + c * _GCH, _GCH)], buf[b],
                              sr[b]).wait()
        pltpu.sync_copy(sel_hbm.at[pl.ds(base + c * _GCH, _GCH)], idx[b])
        pltpu.async_copy(buf[b], out_hbm.at[idx[b]], sw[b])
        n = c + 2
        if n < _NCH:
            bn = n % 3
            if n >= 3:
                pltpu.make_async_copy(buf[bn], out_hbm.at[idx[bn]],
                                      sw[bn]).wait()
            start_read(n, bn)
    for t in range(_NCH - 3, _NCH):
        pltpu.make_async_copy(buf[t % 3], out_hbm.at[idx[t % 3]],
                              sw[t % 3]).wait()


def _scatter_into(out_ref, yg, sel_idx):
    f = pl.kernel(
        _scatter_body,
        out_type=(),
        mesh=plsc.VectorSubcoreMesh(core_axis_name="c", subcore_axis_name="s"),
        scratch_types=(
            [pltpu.VMEM((_GCH,), jnp.int32)] * 3
            + [pltpu.VMEM((_GCH, D), jnp.float32)] * 3
            + [pltpu.SemaphoreType.DMA] * 6
        ),
        compiler_params=pltpu.CompilerParams(needs_layout_passes=False),
    )
    f(yg, sel_idx, out_ref)


def kernel(hidden_states, router_weight, router_bias, W1, b1, W2, b2):
    h_flat = hidden_states.reshape(N, D)
    logits2d, base = _router_logits(h_flat, router_weight, router_bias)
    logits = logits2d.reshape(N)
    sel_idx, sel_w = _select(logits)
    xg = _gather(h_flat, sel_idx)
    yg = _ffn(xg, W1.astype(jnp.bfloat16), b1, W2.astype(jnp.bfloat16), b2,
              sel_w)
    out_ref = jax.new_ref(base)
    _scatter_into(out_ref, yg, sel_idx)
    return out_ref[...].reshape(B, S, D)
